# Initial kernel scaffold; baseline (speedup 1.0000x reference)
import jax
import jax.numpy as jnp
from jax.experimental import pallas as pl


def kernel(x, sto_sub, sto_all, log_k, W1, b1, W2, b2, W3, b3, W4, b4, met_sub, rxn_sub, met_all, rxn_all):
    raise NotImplementedError("write your pallas kernel here")


# speedup vs baseline: 11.5843x; 11.5843x over previous
import functools

import jax
import jax.numpy as jnp
from jax import lax
from jax.experimental import pallas as pl
from jax.experimental.pallas import tpu as pltpu, tpu_sc as plsc

N_MET = 50000
N_RXN = 50000
E_SUB = 800000
E_ALL = 1600000
HIDDEN = 64
MSG_DIM = 32

N_PAD = 50176
N_ACC = 50432
SLICE = N_ACC // 16
E1P = 819200
E2P = 1638400
CHUNK = 2560

_mesh = plsc.VectorSubcoreMesh(core_axis_name="c", subcore_axis_name="s")
_sc_params = pltpu.CompilerParams(needs_layout_passes=False,
                                  use_tc_tiling_on_sc=False)


def _wid():
    return lax.axis_index("s") * 2 + lax.axis_index("c")


@functools.partial(
    pl.kernel, mesh=_mesh, compiler_params=_sc_params,
    out_type=[jax.ShapeDtypeStruct((E1P,), jnp.float32),
              jax.ShapeDtypeStruct((2 * N_ACC,), jnp.float32),
              jax.ShapeDtypeStruct((2 * N_ACC,), jnp.float32)],
    scratch_types=[pltpu.VMEM((N_MET,), jnp.float32),
                   pltpu.VMEM((N_MET,), jnp.float32),
                   pltpu.VMEM((CHUNK,), jnp.int32),
                   pltpu.VMEM((CHUNK,), jnp.int32),
                   pltpu.VMEM((CHUNK,), jnp.float32),
                   pltpu.VMEM((CHUNK,), jnp.float32),
                   pltpu.VMEM((CHUNK,), jnp.float32),
                   pltpu.VMEM((SLICE,), jnp.float32),
                   pltpu.VMEM_SHARED((N_ACC,), jnp.float32),
                   pltpu.VMEM_SHARED((N_ACC,), jnp.float32)])
def _kern_a(conc_h, ext_h, met_h, rxn_h, ones_h, z_h,
            c_out, ext_out, cnt_out,
            conc_t, ext_t, met_v, rxn_v, c_v, ev, ones_v, zb,
            ext_acc, cnt_acc):
    c = lax.axis_index("c")
    s = lax.axis_index("s")
    wid = _wid()
    pltpu.sync_copy(conc_h, conc_t)
    pltpu.sync_copy(ext_h, ext_t)
    pltpu.sync_copy(ones_h, ones_v)
    r0 = s * SLICE
    pltpu.sync_copy(z_h.at[pl.ds(r0, SLICE)], zb)
    pltpu.sync_copy(zb, ext_acc.at[pl.ds(r0, SLICE)])
    pltpu.sync_copy(zb, cnt_acc.at[pl.ds(r0, SLICE)])
    plsc.subcore_barrier()

    def chunk(ch, _):
        ebase = wid * 25600 + ch * CHUNK
        pltpu.sync_copy(met_h.at[pl.ds(ebase, CHUNK)], met_v)
        pltpu.sync_copy(rxn_h.at[pl.ds(ebase, CHUNK)], rxn_v)

        def g(i, _):
            idx = met_v[pl.ds(i * 16, 16)]
            c_v[pl.ds(i * 16, 16)] = plsc.load_gather(conc_t, [idx])
            ev[pl.ds(i * 16, 16)] = plsc.load_gather(ext_t, [idx])
            return 0

        lax.fori_loop(0, CHUNK // 16, g, 0)
        pltpu.sync_copy(c_v, c_out.at[pl.ds(ebase, CHUNK)])
        pltpu.sync_copy(ev, ext_acc.at[rxn_v], add=True)
        pltpu.sync_copy(ones_v, cnt_acc.at[rxn_v], add=True)
        return 0

    lax.fori_loop(0, 10, chunk, 0)
    plsc.subcore_barrier()
    pltpu.sync_copy(ext_acc.at[pl.ds(r0, SLICE)], zb)
    pltpu.sync_copy(zb, ext_out.at[pl.ds(c * N_ACC + r0, SLICE)])
    pltpu.sync_copy(cnt_acc.at[pl.ds(r0, SLICE)], zb)
    pltpu.sync_copy(zb, cnt_out.at[pl.ds(c * N_ACC + r0, SLICE)])


HALF = MSG_DIM // 2


@functools.partial(
    pl.kernel, mesh=_mesh, compiler_params=_sc_params,
    out_type=[jax.ShapeDtypeStruct((2 * N_ACC, HALF), jnp.float32)],
    scratch_types=[pltpu.VMEM((CHUNK, HALF), jnp.float32),
                   pltpu.VMEM((CHUNK,), jnp.int32),
                   pltpu.VMEM((800, HALF), jnp.float32),
                   pltpu.VMEM_SHARED((N_ACC, HALF), jnp.float32)])
def _kern_b2(mlo_h, mhi_h, rxn_h, z2_h, hacc_out, msgb, rxn_v, rb, acc):
    c = lax.axis_index("c")
    s = lax.axis_index("s")
    wid = s
    r0 = s * SLICE
    for off, sz in ((0, 800), (800, 800), (1600, 800), (2400, 752)):
        pltpu.sync_copy(z2_h.at[pl.ds(r0 + off, sz)], rb.at[pl.ds(0, sz)])
        pltpu.sync_copy(rb.at[pl.ds(0, sz)], acc.at[pl.ds(r0 + off, sz)])
    plsc.subcore_barrier()

    def do_half(msg_half):
        def chunk(ch, _):
            ebase = wid * 51200 + ch * CHUNK
            pltpu.sync_copy(msg_half.at[pl.ds(ebase, CHUNK)], msgb)
            pltpu.sync_copy(rxn_h.at[pl.ds(ebase, CHUNK)], rxn_v)
            pltpu.sync_copy(msgb, acc.at[rxn_v], add=True)
            return 0
        lax.fori_loop(0, 20, chunk, 0)

    @pl.when(c == 0)
    def _():
        do_half(mlo_h)

    @pl.when(c == 1)
    def _():
        do_half(mhi_h)

    plsc.subcore_barrier()
    for off, sz in ((0, 800), (800, 800), (1600, 800), (2400, 752)):
        pltpu.sync_copy(acc.at[pl.ds(r0 + off, sz)], rb.at[pl.ds(0, sz)])
        pltpu.sync_copy(rb.at[pl.ds(0, sz)],
                        hacc_out.at[pl.ds(c * N_ACC + r0 + off, sz)])


@functools.partial(
    pl.kernel, mesh=_mesh, compiler_params=_sc_params,
    out_type=[jax.ShapeDtypeStruct((2 * N_ACC,), jnp.float32)],
    scratch_types=[pltpu.VMEM((N_PAD,), jnp.float32),
                   pltpu.VMEM((CHUNK,), jnp.int32),
                   pltpu.VMEM((CHUNK,), jnp.int32),
                   pltpu.VMEM((CHUNK,), jnp.float32),
                   pltpu.VMEM((CHUNK,), jnp.float32),
                   pltpu.VMEM((SLICE,), jnp.float32),
                   pltpu.VMEM_SHARED((N_ACC,), jnp.float32)])
def _kern_d(v_h, rall_h, sall_h, mall_h, z_h, dx_out,
            vt, rx_v, mt_v, st_v, pr_v, zb, acc):
    c = lax.axis_index("c")
    s = lax.axis_index("s")
    wid = _wid()
    pltpu.sync_copy(v_h, vt)
    r0 = s * SLICE
    pltpu.sync_copy(z_h.at[pl.ds(r0, SLICE)], zb)
    pltpu.sync_copy(zb, acc.at[pl.ds(r0, SLICE)])
    plsc.subcore_barrier()

    def chunk(ch, _):
        ebase = wid * 51200 + ch * CHUNK
        pltpu.sync_copy(rall_h.at[pl.ds(ebase, CHUNK)], rx_v)
        pltpu.sync_copy(mall_h.at[pl.ds(ebase, CHUNK)], mt_v)
        pltpu.sync_copy(sall_h.at[pl.ds(ebase, CHUNK)], st_v)

        def g(i, _):
            idx = rx_v[pl.ds(i * 16, 16)]
            vv = plsc.load_gather(vt, [idx])
            pr_v[pl.ds(i * 16, 16)] = vv * st_v[pl.ds(i * 16, 16)]
            return 0

        lax.fori_loop(0, CHUNK // 16, g, 0)
        pltpu.sync_copy(pr_v, acc.at[mt_v], add=True)
        return 0

    lax.fori_loop(0, 20, chunk, 0)
    plsc.subcore_barrier()
    pltpu.sync_copy(acc.at[pl.ds(r0, SLICE)], zb)
    pltpu.sync_copy(zb, dx_out.at[pl.ds(c * N_ACC + r0, SLICE)])


def _b1_body(c_ref, s_ref, w1_ref, b1_ref, w2_ref, b2_ref, lo_ref, hi_ref):
    cv = c_ref[...]
    sv = s_ref[...]
    h = jnp.tanh(cv * w1_ref[0:1, :] + sv * w1_ref[1:2, :] + b1_ref[...])
    m = jnp.dot(h, w2_ref[...],
                preferred_element_type=jnp.float32) + b2_ref[...]
    lo_ref[...] = m[:, :HALF]
    hi_ref[...] = m[:, HALF:]


def _run_b1(c_e, ssub, W1, b1, W2, b2):
    B = 2048
    grid = E1P // B
    return pl.pallas_call(
        _b1_body,
        grid=(grid,),
        in_specs=[pl.BlockSpec((B, 1), lambda i: (i, 0)),
                  pl.BlockSpec((B, 1), lambda i: (i, 0)),
                  pl.BlockSpec((2, HIDDEN), lambda i: (0, 0)),
                  pl.BlockSpec((1, HIDDEN), lambda i: (0, 0)),
                  pl.BlockSpec((HIDDEN, MSG_DIM), lambda i: (0, 0)),
                  pl.BlockSpec((1, MSG_DIM), lambda i: (0, 0))],
        out_specs=[pl.BlockSpec((B, HALF), lambda i: (i, 0)),
                   pl.BlockSpec((B, HALF), lambda i: (i, 0))],
        out_shape=[jax.ShapeDtypeStruct((E1P, HALF), jnp.float32),
                   jax.ShapeDtypeStruct((E1P, HALF), jnp.float32)],
    )(c_e.reshape(E1P, 1), ssub.reshape(E1P, 1), W1,
      b1.reshape(1, HIDDEN), W2, b2.reshape(1, MSG_DIM))


def _b3_body(h0_ref, h1_ref, e0_ref, e1_ref, c0_ref, c1_ref, lk_ref,
             w3_ref, b3_ref, w4_ref, b4_ref, v_ref):
    h = jnp.concatenate([h0_ref[...], h1_ref[...]], axis=1)
    hr = jnp.tanh(jnp.dot(h, w3_ref[...],
                          preferred_element_type=jnp.float32) + b3_ref[...])
    z = jnp.dot(hr, w4_ref[...],
                preferred_element_type=jnp.float32) + b4_ref[...]
    base_v = jnp.maximum(z, 0.0) + jnp.log1p(jnp.exp(-jnp.abs(z)))
    cnt = jnp.maximum(c0_ref[...] + c1_ref[...], 1.0)
    ext_mean = (e0_ref[...] + e1_ref[...]) / cnt
    k10 = jnp.exp(lk_ref[...] * 2.302585092994046)
    v_ref[...] = k10 * ext_mean * base_v


def _run_b3(h0, h1, e0, e1, c0, c1, lk, W3, b3, W4, b4):
    B = 256
    grid = N_PAD // B
    spec1 = pl.BlockSpec((B, 1), lambda i: (i, 0))
    return pl.pallas_call(
        _b3_body,
        grid=(grid,),
        in_specs=[pl.BlockSpec((B, HALF), lambda i: (i, 0)),
                  pl.BlockSpec((B, HALF), lambda i: (i, 0)),
                  spec1, spec1, spec1, spec1, spec1,
                  pl.BlockSpec((MSG_DIM, HIDDEN), lambda i: (0, 0)),
                  pl.BlockSpec((1, HIDDEN), lambda i: (0, 0)),
                  pl.BlockSpec((HIDDEN, 1), lambda i: (0, 0)),
                  pl.BlockSpec((1, 1), lambda i: (0, 0))],
        out_specs=spec1,
        out_shape=jax.ShapeDtypeStruct((N_PAD, 1), jnp.float32),
    )(h0, h1, e0, e1, c0, c1, lk, W3,
      b3.reshape(1, HIDDEN), W4, b4.reshape(1, 1))


def _e_body(d0_ref, d1_ref, o_ref):
    o_ref[...] = 0.005 * (d0_ref[...] + d1_ref[...])


def _run_e(d0, d1):
    B = 256
    spec = pl.BlockSpec((B, 1), lambda i: (i, 0))
    return pl.pallas_call(
        _e_body,
        grid=(N_PAD // B,),
        in_specs=[spec, spec],
        out_specs=spec,
        out_shape=jax.ShapeDtypeStruct((N_PAD, 1), jnp.float32),
    )(d0, d1)


def kernel(x, sto_sub, sto_all, log_k, W1, b1, W2, b2, W3, b3, W4, b4,
           met_sub, rxn_sub, met_all, rxn_all):
    f32 = jnp.float32
    i32 = jnp.int32
    conc = x[:, 3]
    ext = x[:, 4]

    p1 = E1P - E_SUB
    sent1 = N_PAD + (jnp.arange(p1, dtype=i32) % 64)
    msub = jnp.concatenate([met_sub.astype(i32), jnp.zeros((p1,), i32)])
    rsub = jnp.concatenate([rxn_sub.astype(i32), sent1])
    ssub = jnp.concatenate([sto_sub, jnp.zeros((p1,), f32)])

    ones1 = jnp.ones((CHUNK,), f32)
    z1 = jnp.zeros((N_ACC,), f32)
    z2 = jnp.zeros((N_ACC, HALF), f32)

    c_e, ext_sums, cnt_sums = _kern_a(conc, ext, msub, rsub, ones1, z1)

    mlo, mhi = _run_b1(c_e, ssub, W1, b1, W2, b2)

    (hacc,) = _kern_b2(mlo, mhi, rsub, z2)

    lk = jnp.concatenate([log_k, jnp.zeros((N_PAD - N_RXN,), f32)])
    v = _run_b3(hacc[:N_PAD], hacc[N_ACC:N_ACC + N_PAD],
                ext_sums[:N_PAD].reshape(-1, 1),
                ext_sums[N_ACC:N_ACC + N_PAD].reshape(-1, 1),
                cnt_sums[:N_PAD].reshape(-1, 1),
                cnt_sums[N_ACC:N_ACC + N_PAD].reshape(-1, 1),
                lk.reshape(-1, 1), W3, b3, W4, b4)

    p2 = E2P - E_ALL
    sent2 = N_PAD + (jnp.arange(p2, dtype=i32) % 64)
    rall = jnp.concatenate([rxn_all.astype(i32), jnp.zeros((p2,), i32)])
    mall = jnp.concatenate([met_all.astype(i32), sent2])
    sall = jnp.concatenate([sto_all, jnp.zeros((p2,), f32)])

    (dx,) = _kern_d(v.reshape(N_PAD), rall, sall, mall, z1)

    out = _run_e(dx[:N_PAD].reshape(-1, 1),
                 dx[N_ACC:N_ACC + N_PAD].reshape(-1, 1))
    return out[:N_MET]


# speedup vs baseline: 27.8843x; 2.4071x over previous
import functools

import jax
import jax.numpy as jnp
from jax import lax
from jax.experimental import pallas as pl
from jax.experimental.pallas import tpu as pltpu, tpu_sc as plsc

N_MET = 50000
N_RXN = 50000
E_SUB = 800000
E_ALL = 1600000
HIDDEN = 64
MSG_DIM = 32

N_PAD = 50176
N_ACC = 50432
SLICE = N_ACC // 16
E1P = 819200
E2P = 1638400
CHUNK = 2560

_mesh = plsc.VectorSubcoreMesh(core_axis_name="c", subcore_axis_name="s")
_sc_params = pltpu.CompilerParams(needs_layout_passes=False,
                                  use_tc_tiling_on_sc=False)


def _wid():
    return lax.axis_index("s") * 2 + lax.axis_index("c")


@functools.partial(
    pl.kernel, mesh=_mesh, compiler_params=_sc_params,
    out_type=[jax.ShapeDtypeStruct((E1P,), jnp.float32),
              jax.ShapeDtypeStruct((E1P,), jnp.float32)],
    scratch_types=[pltpu.VMEM((N_MET,), jnp.float32),
                   pltpu.VMEM((N_MET,), jnp.float32),
                   pltpu.VMEM((CHUNK,), jnp.int32),
                   pltpu.VMEM((CHUNK,), jnp.float32),
                   pltpu.VMEM((CHUNK,), jnp.float32)])
def _kern_a(conc_h, ext_h, met_h, c_out, e_out,
            conc_t, ext_t, met_v, c_v, ev):
    c = lax.axis_index("c")
    s = lax.axis_index("s")
    wid = _wid()
    pltpu.sync_copy(conc_h, conc_t)
    pltpu.sync_copy(ext_h, ext_t)

    def chunk(ch, _):
        ebase = wid * 25600 + ch * CHUNK
        pltpu.sync_copy(met_h.at[pl.ds(ebase, CHUNK)], met_v)

        def g(i, _):
            idx = met_v[pl.ds(i * 16, 16)]
            c_v[pl.ds(i * 16, 16)] = plsc.load_gather(conc_t, [idx])
            ev[pl.ds(i * 16, 16)] = plsc.load_gather(ext_t, [idx])
            return 0

        lax.fori_loop(0, CHUNK // 16, g, 0)
        pltpu.sync_copy(c_v, c_out.at[pl.ds(ebase, CHUNK)])
        pltpu.sync_copy(ev, e_out.at[pl.ds(ebase, CHUNK)])
        return 0

    lax.fori_loop(0, 10, chunk, 0)


@functools.partial(
    pl.kernel, mesh=_mesh, compiler_params=_sc_params,
    out_type=[jax.ShapeDtypeStruct((32 * N_ACC,), jnp.float32),
              jax.ShapeDtypeStruct((32 * N_ACC,), jnp.float32)],
    scratch_types=[pltpu.VMEM((CHUNK,), jnp.float32),
                   pltpu.VMEM((CHUNK,), jnp.int32),
                   pltpu.VMEM((N_ACC,), jnp.float32),
                   pltpu.VMEM((N_ACC,), jnp.float32)])
def _kern_a2(ee_h, rxn_h, zt_h, ext_out, cnt_out,
             ev, rxn_v, ext_acc, cnt_acc):
    c = lax.axis_index("c")
    s = lax.axis_index("s")
    wid = _wid()
    pltpu.sync_copy(zt_h, ext_acc)
    pltpu.sync_copy(zt_h, cnt_acc)
    iota = lax.iota(jnp.int32, 16)
    ones = jnp.full((16,), 1.0, jnp.float32)

    def chunk(ch, _):
        ebase = wid * 25600 + ch * CHUNK
        pltpu.sync_copy(ee_h.at[pl.ds(ebase, CHUNK)], ev)
        pltpu.sync_copy(rxn_h.at[pl.ds(ebase, CHUNK)], rxn_v)

        def g(i, _):
            idx = rxn_v[pl.ds(i * 16, 16)]
            plsc.addupdate_scatter(ext_acc, [idx], ev[pl.ds(i * 16, 16)])
            plsc.addupdate_scatter(cnt_acc, [idx], ones)
            return 0

        lax.fori_loop(0, CHUNK // 16, g, 0)
        return 0

    lax.fori_loop(0, 10, chunk, 0)
    pltpu.sync_copy(ext_acc, ext_out.at[pl.ds(wid * N_ACC, N_ACC)])
    pltpu.sync_copy(cnt_acc, cnt_out.at[pl.ds(wid * N_ACC, N_ACC)])


HALF = MSG_DIM // 2
EROWS = E1P // 128


@functools.partial(
    pl.kernel, mesh=_mesh, compiler_params=_sc_params,
    out_type=[jax.ShapeDtypeStruct((2, 2 * N_ACC, 8), jnp.float32)],
    scratch_types=[pltpu.VMEM((8, 20, 128), jnp.float32),
                   pltpu.VMEM((CHUNK, 8), jnp.float32),
                   pltpu.VMEM((CHUNK,), jnp.int32),
                   pltpu.VMEM((800, 8), jnp.float32),
                   pltpu.VMEM_SHARED((N_ACC, 8), jnp.float32)])
def _kern_b2(msgt_h, rxn_h, z2_h, hacc_out, mb, msgb2, rxn_v, rb, acc):
    c = lax.axis_index("c")
    s = lax.axis_index("s")
    wid = s
    r0 = s * SLICE
    iota = lax.iota(jnp.int32, 16)

    for q in range(2):
        for off, sz in ((0, 800), (800, 800), (1600, 800), (2400, 752)):
            pltpu.sync_copy(z2_h.at[pl.ds(r0 + off, sz)], rb.at[pl.ds(0, sz)])
            pltpu.sync_copy(rb.at[pl.ds(0, sz)], acc.at[pl.ds(r0 + off, sz)])
        plsc.subcore_barrier()

        def chunk(ch, _):
            ebase = wid * 51200 + ch * CHUNK
            jb = ebase // 128
            pltpu.sync_copy(
                msgt_h.at[c, pl.ds(q * 8, 8), pl.ds(jb, 20)], mb)
            pltpu.sync_copy(rxn_h.at[pl.ds(ebase, CHUNK)], rxn_v)

            def g(i, _):
                rows = i * 16 + iota
                j = lax.shift_right_logical(i, 3)
                lane0 = lax.bitwise_and(i, 7) * 16
                for r in range(8):
                    vec = mb[r, j, pl.ds(lane0, 16)]
                    plsc.store_scatter(
                        msgb2, [rows, jnp.full((16,), r, jnp.int32)], vec)
                return 0

            lax.fori_loop(0, CHUNK // 16, g, 0)
            pltpu.sync_copy(msgb2, acc.at[rxn_v], add=True)
            return 0

        lax.fori_loop(0, 20, chunk, 0)
        plsc.subcore_barrier()
        for off, sz in ((0, 800), (800, 800), (1600, 800), (2400, 752)):
            pltpu.sync_copy(acc.at[pl.ds(r0 + off, sz)], rb.at[pl.ds(0, sz)])
            pltpu.sync_copy(
                rb.at[pl.ds(0, sz)],
                hacc_out.at[q, pl.ds(c * N_ACC + r0 + off, sz)])
        plsc.subcore_barrier()


@functools.partial(
    pl.kernel, mesh=_mesh, compiler_params=_sc_params,
    out_type=[jax.ShapeDtypeStruct((32 * N_ACC,), jnp.float32)],
    scratch_types=[pltpu.VMEM((N_PAD,), jnp.float32),
                   pltpu.VMEM((CHUNK,), jnp.int32),
                   pltpu.VMEM((CHUNK,), jnp.int32),
                   pltpu.VMEM((CHUNK,), jnp.float32),
                   pltpu.VMEM((N_ACC,), jnp.float32)])
def _kern_d(v_h, rall_h, sall_h, mall_h, zt_h, dx_out,
            vt, rx_v, mt_v, st_v, acc):
    c = lax.axis_index("c")
    s = lax.axis_index("s")
    wid = _wid()
    pltpu.sync_copy(v_h, vt)
    pltpu.sync_copy(zt_h, acc)

    def chunk(ch, _):
        ebase = wid * 51200 + ch * CHUNK
        pltpu.sync_copy(rall_h.at[pl.ds(ebase, CHUNK)], rx_v)
        pltpu.sync_copy(mall_h.at[pl.ds(ebase, CHUNK)], mt_v)
        pltpu.sync_copy(sall_h.at[pl.ds(ebase, CHUNK)], st_v)

        def g(i, _):
            idx = rx_v[pl.ds(i * 16, 16)]
            vv = plsc.load_gather(vt, [idx])
            plsc.addupdate_scatter(acc, [mt_v[pl.ds(i * 16, 16)]],
                                   vv * st_v[pl.ds(i * 16, 16)])
            return 0

        lax.fori_loop(0, CHUNK // 16, g, 0)
        return 0

    lax.fori_loop(0, 20, chunk, 0)
    pltpu.sync_copy(acc, dx_out.at[pl.ds(wid * N_ACC, N_ACC)])


def _b1_body(c_ref, s_ref, w1t_ref, b1_ref, w2t_ref, b2_ref, o_ref):
    w1a = w1t_ref[:, 0:1]
    w1b = w1t_ref[:, 1:2]
    for r in range(8):
        cv = c_ref[r:r + 1, :]
        sv = s_ref[r:r + 1, :]
        ht = jnp.tanh(w1a * cv + w1b * sv + b1_ref[...])
        mt = jnp.dot(w2t_ref[...], ht,
                     preferred_element_type=jnp.float32) + b2_ref[...]
        o_ref[:, :, r * 16:(r + 1) * 16, :] = mt.reshape(2, 16, 16, 128)


def _run_b1(c_e, ssub, W1, b1, W2, b2):
    R = E1P // 2048
    return pl.pallas_call(
        _b1_body,
        grid=(R // 8,),
        in_specs=[pl.BlockSpec((8, 2048), lambda i: (i, 0)),
                  pl.BlockSpec((8, 2048), lambda i: (i, 0)),
                  pl.BlockSpec((HIDDEN, 2), lambda i: (0, 0)),
                  pl.BlockSpec((HIDDEN, 1), lambda i: (0, 0)),
                  pl.BlockSpec((MSG_DIM, HIDDEN), lambda i: (0, 0)),
                  pl.BlockSpec((MSG_DIM, 1), lambda i: (0, 0))],
        out_specs=pl.BlockSpec((2, 16, 8 * 16, 128), lambda i: (0, 0, i, 0)),
        out_shape=jax.ShapeDtypeStruct((2, 16, EROWS, 128), jnp.float32),
    )(c_e.reshape(R, 2048), ssub.reshape(R, 2048), W1.T,
      b1.reshape(HIDDEN, 1), W2.T, b2.reshape(MSG_DIM, 1))


def _b3a_body(h_ref, w3_ref, b3_ref, w4_ref, b4_ref, z_ref):
    h = h_ref[...]
    hr = jnp.tanh(jnp.dot(h, w3_ref[...],
                          preferred_element_type=jnp.float32) + b3_ref[...])
    z_ref[...] = jnp.dot(hr, w4_ref[...],
                         preferred_element_type=jnp.float32) + b4_ref[...]


def _run_b3a(h, W3, b3, W4, b4):
    B = 1024
    return pl.pallas_call(
        _b3a_body,
        grid=(N_PAD // B,),
        in_specs=[pl.BlockSpec((B, MSG_DIM), lambda i: (i, 0)),
                  pl.BlockSpec((MSG_DIM, HIDDEN), lambda i: (0, 0)),
                  pl.BlockSpec((1, HIDDEN), lambda i: (0, 0)),
                  pl.BlockSpec((HIDDEN, 1), lambda i: (0, 0)),
                  pl.BlockSpec((1, 1), lambda i: (0, 0))],
        out_specs=pl.BlockSpec((B, 1), lambda i: (i, 0)),
        out_shape=jax.ShapeDtypeStruct((N_PAD, 1), jnp.float32),
    )(h, W3, b3.reshape(1, HIDDEN), W4, b4.reshape(1, 1))


def _b3b_body(z_ref, e_ref, c_ref, lk_ref, v_ref):
    z = z_ref[...]
    base_v = jnp.maximum(z, 0.0) + jnp.log1p(jnp.exp(-jnp.abs(z)))
    cnt = jnp.maximum(jnp.sum(c_ref[...], axis=0), 1.0)
    ext_mean = jnp.sum(e_ref[...], axis=0) / cnt
    k10 = jnp.exp(lk_ref[...] * 2.302585092994046)
    v_ref[...] = k10 * ext_mean * base_v


def _run_b3b(z2d, e32, c32, lk):
    spec = pl.BlockSpec((8, 128), lambda i: (i, 0))
    spec32 = pl.BlockSpec((32, 8, 128), lambda i: (0, i, 0))
    return pl.pallas_call(
        _b3b_body,
        grid=(N_PAD // 1024,),
        in_specs=[spec, spec32, spec32, spec],
        out_specs=spec,
        out_shape=jax.ShapeDtypeStruct((N_PAD // 128, 128), jnp.float32),
    )(z2d, e32, c32, lk)


def _e_body(d_ref, o_ref):
    o_ref[...] = 0.005 * jnp.sum(d_ref[...], axis=0)


def _run_e(d32):
    return pl.pallas_call(
        _e_body,
        grid=(N_PAD // 1024,),
        in_specs=[pl.BlockSpec((32, 8, 128), lambda i: (0, i, 0))],
        out_specs=pl.BlockSpec((8, 128), lambda i: (i, 0)),
        out_shape=jax.ShapeDtypeStruct((N_PAD // 128, 128), jnp.float32),
    )(d32)


def kernel(x, sto_sub, sto_all, log_k, W1, b1, W2, b2, W3, b3, W4, b4,
           met_sub, rxn_sub, met_all, rxn_all):
    f32 = jnp.float32
    i32 = jnp.int32
    conc = x[:, 3]
    ext = x[:, 4]

    p1 = E1P - E_SUB
    sent1 = N_PAD + (jnp.arange(p1, dtype=i32) % 64)
    msub = jnp.concatenate([met_sub.astype(i32), jnp.zeros((p1,), i32)])
    rsub = jnp.concatenate([rxn_sub.astype(i32), sent1])
    ssub = jnp.concatenate([sto_sub, jnp.zeros((p1,), f32)])

    z1 = jnp.zeros((N_ACC,), f32)
    z2 = jnp.zeros((N_ACC, 8), f32)

    c_e, e_e = _kern_a(conc, ext, msub)
    ext_32, cnt_32 = _kern_a2(e_e, rsub, z1)

    msgt = _run_b1(c_e, ssub, W1, b1, W2, b2)

    (hacc,) = _kern_b2(msgt, rsub, z2)

    lk = jnp.concatenate([log_k, jnp.zeros((N_PAD - N_RXN,), f32)])
    h_all = jnp.concatenate(
        [hacc[0, :N_PAD], hacc[1, :N_PAD],
         hacc[0, N_ACC:N_ACC + N_PAD], hacc[1, N_ACC:N_ACC + N_PAD]], axis=1)
    z = _run_b3a(h_all, W3, b3, W4, b4)
    e32 = ext_32.reshape(32, N_ACC)[:, :N_PAD].reshape(32, -1, 128)
    c32 = cnt_32.reshape(32, N_ACC)[:, :N_PAD].reshape(32, -1, 128)
    v = _run_b3b(z.reshape(-1, 128), e32, c32, lk.reshape(-1, 128))

    p2 = E2P - E_ALL
    sent2 = N_PAD + (jnp.arange(p2, dtype=i32) % 64)
    rall = jnp.concatenate([rxn_all.astype(i32), jnp.zeros((p2,), i32)])
    mall = jnp.concatenate([met_all.astype(i32), sent2])
    sall = jnp.concatenate([sto_all, jnp.zeros((p2,), f32)])

    zt = jnp.zeros((N_ACC,), f32)
    (dx,) = _kern_d(v.reshape(N_PAD), rall, sall, mall, zt)

    d32 = dx.reshape(32, N_ACC)[:, :N_PAD].reshape(32, -1, 128)
    out = _run_e(d32)
    return out.reshape(N_PAD, 1)[:N_MET]
